# Initial kernel scaffold; baseline (speedup 1.0000x reference)
import jax
import jax.numpy as jnp
from jax.experimental import pallas as pl


def kernel(x, table):
    raise NotImplementedError("write your pallas kernel here")


# speedup vs baseline: 2.4607x; 2.4607x over previous
import functools

import jax
import jax.numpy as jnp
from jax import lax
from jax.experimental import pallas as pl
from jax.experimental.pallas import tpu as pltpu
from jax.experimental.pallas import tpu_sc as plsc

_BUCKETS = (1, 2, 3, 4, 5, 8, 16, 32, 64)

_N = 1_000_000
_D = 20
_V = 128
_C = 2000
_G = _N // _C
_L = 16


def _body(x_hbm, table_hbm, out_hbm, table_v, lut_v, x_v, out_v):
    nc = 2
    wid = lax.axis_index("s") * nc + lax.axis_index("c")
    nw = 32

    pltpu.sync_copy(table_hbm, table_v)

    for g in range(_V // _L):
        val = lax.iota(jnp.int32, _L) + g * _L
        idx = jnp.zeros((_L,), jnp.int32)
        for b in _BUCKETS:
            idx = idx + jnp.where(val >= b, 1, 0)
        for c in range(_D):
            colv = jnp.full((_L,), c, jnp.int32)
            w = plsc.load_gather(table_v, [idx, colv])
            plsc.store_scatter(lut_v, [val, colv], w)

    colvs = [jnp.full((_L,), c, jnp.int32) for c in range(_D)]

    def chunk_body(i, carry):
        g = wid + i * nw

        @pl.when(g < _G)
        def _():
            base = g * _C
            pltpu.sync_copy(x_hbm.at[pl.ds(base, _C)], x_v)

            def group(j, carry2):
                off = pl.multiple_of(j * _L, _L)
                xv = x_v[pl.ds(off, _L)]
                rows = lax.iota(jnp.int32, _L) + j * _L
                for c in range(_D):
                    w = plsc.load_gather(lut_v, [xv, colvs[c]])
                    plsc.store_scatter(out_v, [rows, colvs[c]], w)
                return carry2

            lax.fori_loop(0, _C // _L, group, 0)
            pltpu.sync_copy(out_v, out_hbm.at[pl.ds(base, _C)])

        return carry

    lax.fori_loop(0, (_G + nw - 1) // nw, chunk_body, 0)


@jax.jit
def _sc_embed(x, table):
    mesh = plsc.VectorSubcoreMesh(core_axis_name="c", subcore_axis_name="s")
    fn = pl.kernel(
        _body,
        out_type=jax.ShapeDtypeStruct((_N, _D), jnp.float32),
        mesh=mesh,
        scratch_types=[
            pltpu.VMEM((10, _D), jnp.float32),
            pltpu.VMEM((_V, _D), jnp.float32),
            pltpu.VMEM((_C,), jnp.int32),
            pltpu.VMEM((_C, _D), jnp.float32),
        ],
        compiler_params=pltpu.CompilerParams(
            needs_layout_passes=False, use_tc_tiling_on_sc=False
        ),
    )
    return fn(x, table)


def kernel(x, table):
    return _sc_embed(x, table)


# speedup vs baseline: 2.7313x; 1.1100x over previous
import functools

import jax
import jax.numpy as jnp
from jax import lax
from jax.experimental import pallas as pl
from jax.experimental.pallas import tpu as pltpu
from jax.experimental.pallas import tpu_sc as plsc

_BUCKETS = (1, 2, 3, 4, 5, 8, 16, 32, 64)

_N = 1_000_000
_D = 20
_V = 128
_C = 2000
_G = _N // _C
_L = 16
_NW = 32


def _body(x_hbm, table_hbm, out_hbm, table_v, lut_v, x_v, out_v0, out_v1,
          sem0, sem1):
    wid = lax.axis_index("s") * 2 + lax.axis_index("c")

    pltpu.sync_copy(table_hbm, table_v)

    colvs = [jnp.full((_L,), c, jnp.int32) for c in range(_D)]

    for g in range(_V // _L):
        val = lax.iota(jnp.int32, _L) + g * _L
        idx = jnp.zeros((_L,), jnp.int32)
        for b in _BUCKETS:
            idx = idx + jnp.where(val >= b, 1, 0)
        for c in range(_D):
            w = plsc.load_gather(table_v, [idx, colvs[c]])
            plsc.store_scatter(lut_v, [val, colvs[c]], w)

    bufs = (out_v0, out_v1)
    sems = (sem0, sem1)

    def pair_body(i, carry):
        for b in range(2):
            g = wid + (2 * i + b) * _NW

            @pl.when(g < _G)
            def _():
                base = g * _C
                pltpu.sync_copy(x_hbm.at[pl.ds(base, _C)], x_v)

                @pl.when(i >= 1)
                def _():
                    pltpu.make_async_copy(
                        bufs[b],
                        out_hbm.at[pl.ds((g - 2 * _NW) * _C, _C)],
                        sems[b],
                    ).wait()

                @plsc.parallel_loop(0, _C // _L, unroll=5)
                def group(j):
                    off = pl.multiple_of(j * _L, _L)
                    xv = x_v[pl.ds(off, _L)]
                    rows = lax.iota(jnp.int32, _L) + j * _L
                    for c in range(_D):
                        w = plsc.load_gather(lut_v, [xv, colvs[c]])
                        plsc.store_scatter(bufs[b], [rows, colvs[c]], w)

                pltpu.async_copy(bufs[b], out_hbm.at[pl.ds(base, _C)], sems[b])

        return carry

    lax.fori_loop(0, (_G + 2 * _NW - 1) // (2 * _NW), pair_body, 0)

    for b in range(2):
        pltpu.make_async_copy(bufs[b], out_hbm.at[pl.ds(b * _C, _C)],
                              sems[b]).wait()


@jax.jit
def _sc_embed(x, table):
    mesh = plsc.VectorSubcoreMesh(core_axis_name="c", subcore_axis_name="s")
    fn = pl.kernel(
        _body,
        out_type=jax.ShapeDtypeStruct((_N, _D), jnp.float32),
        mesh=mesh,
        scratch_types=[
            pltpu.VMEM((10, _D), jnp.float32),
            pltpu.VMEM((_V, _D), jnp.float32),
            pltpu.VMEM((_C,), jnp.int32),
            pltpu.VMEM((_C, _D), jnp.float32),
            pltpu.VMEM((_C, _D), jnp.float32),
            pltpu.SemaphoreType.DMA,
            pltpu.SemaphoreType.DMA,
        ],
        compiler_params=pltpu.CompilerParams(
            needs_layout_passes=False, use_tc_tiling_on_sc=False
        ),
    )
    return fn(x, table)


def kernel(x, table):
    return _sc_embed(x, table)


# speedup vs baseline: 31.4503x; 11.5146x over previous
import functools

import jax
import jax.numpy as jnp
from jax import lax
from jax.experimental import pallas as pl
from jax.experimental.pallas import tpu as pltpu
from jax.experimental.pallas import tpu_sc as plsc

_BUCKETS = (1, 2, 3, 4, 5, 8, 16, 32, 64)

_N = 1_000_000
_D = 20
_V = 128
_W = 1024
_GF = _N // _W
_TAIL = _N - _GF * _W
_L = 16
_NW = 32


def _bucketize(val):
    idx = jnp.zeros((_L,), jnp.int32)
    for b in _BUCKETS:
        idx = idx + jnp.where(val >= b, 1, 0)
    return idx


def _build_lut(tflat_v, lut_v):
    for g in range(_V // _L):
        val = lax.iota(jnp.int32, _L) + g * _L
        idx20 = _bucketize(val) * _D
        for j in range(_D):
            w = plsc.load_gather(tflat_v, [idx20 + j])
            plsc.store_scatter(lut_v, [val + j * 128], w)


def _body(x_hbm, tflat_hbm, out_hbm, tflat_v, lut_v, x_v, buf0, buf1, tbuf,
          tx_v, sem0, sem1):
    wid = lax.axis_index("s") * 2 + lax.axis_index("c")

    pltpu.sync_copy(tflat_hbm, tflat_v)
    _build_lut(tflat_v, lut_v)

    bufs = (buf0, buf1)
    sems = (sem0, sem1)

    def compute_panel(x_ref, buf, n_groups):
        @plsc.parallel_loop(0, n_groups, unroll=4)
        def group(q):
            off = pl.multiple_of(q * _L, _L)
            xv = x_ref[pl.ds(off, _L)]
            for j in range(_D):
                buf[j, pl.ds(off, _L)] = plsc.load_gather(lut_v, [xv + j * 128])

    def pair_body(i, carry):
        for b in range(2):
            g = wid + (2 * i + b) * _NW

            @pl.when(g < _GF)
            def _():
                base = g * _W
                pltpu.sync_copy(x_hbm.at[pl.ds(base, _W)], x_v)

                @pl.when(i >= 1)
                def _():
                    pltpu.make_async_copy(
                        bufs[b],
                        out_hbm.at[:, pl.ds((g - 2 * _NW) * _W, _W)],
                        sems[b],
                    ).wait()

                compute_panel(x_v, bufs[b], _W // _L)
                pltpu.async_copy(bufs[b], out_hbm.at[:, pl.ds(base, _W)],
                                 sems[b])

        return carry

    lax.fori_loop(0, (_GF + 2 * _NW - 1) // (2 * _NW), pair_body, 0)

    @pl.when(wid == _NW - 1)
    def _():
        base = _GF * _W
        pltpu.sync_copy(x_hbm.at[pl.ds(base, _TAIL)], tx_v)
        compute_panel(tx_v, tbuf, _TAIL // _L)
        pltpu.sync_copy(tbuf, out_hbm.at[:, pl.ds(base, _TAIL)])

    for b in range(2):
        pltpu.make_async_copy(bufs[b], out_hbm.at[:, pl.ds(b * _W, _W)],
                              sems[b]).wait()


@jax.jit
def _sc_embed(x, table):
    mesh = plsc.VectorSubcoreMesh(core_axis_name="c", subcore_axis_name="s")
    fn = pl.kernel(
        _body,
        out_type=jax.ShapeDtypeStruct((_D, _N), jnp.float32),
        mesh=mesh,
        scratch_types=[
            pltpu.VMEM((_D * 10,), jnp.float32),
            pltpu.VMEM((_D * _V,), jnp.float32),
            pltpu.VMEM((_W,), jnp.int32),
            pltpu.VMEM((_D, _W), jnp.float32),
            pltpu.VMEM((_D, _W), jnp.float32),
            pltpu.VMEM((_D, _TAIL), jnp.float32),
            pltpu.VMEM((_TAIL,), jnp.int32),
            pltpu.SemaphoreType.DMA,
            pltpu.SemaphoreType.DMA,
        ],
        compiler_params=pltpu.CompilerParams(
            needs_layout_passes=False, use_tc_tiling_on_sc=True
        ),
    )
    out_t = fn(x, table.reshape(-1))
    return out_t.T


def kernel(x, table):
    return _sc_embed(x, table)


# speedup vs baseline: 36.2985x; 1.1542x over previous
import functools

import jax
import jax.numpy as jnp
from jax import lax
from jax.experimental import pallas as pl
from jax.experimental.pallas import tpu as pltpu
from jax.experimental.pallas import tpu_sc as plsc

_BUCKETS = (1, 2, 3, 4, 5, 8, 16, 32, 64)

_N = 1_000_000
_D = 20
_V = 128
_W = 1024
_GF = _N // _W
_TAIL = _N - _GF * _W
_L = 16
_NW = 32


def _bucketize(val):
    idx = jnp.zeros((_L,), jnp.int32)
    for b in _BUCKETS:
        idx = idx + jnp.where(val >= b, 1, 0)
    return idx


def _body(x_hbm, tflat_hbm, out_hbm, tflat_v, lut_v, x0, x1, buf0, buf1,
          tbuf, tx_v, sem0, sem1, xsem0, xsem1):
    wid = lax.axis_index("s") * 2 + lax.axis_index("c")

    pltpu.sync_copy(tflat_hbm, tflat_v)

    for g in range(_V // _L):
        val = lax.iota(jnp.int32, _L) + g * _L
        idx20 = _bucketize(val) * _D
        for j in range(_D):
            w = plsc.load_gather(tflat_v, [idx20 + j])
            plsc.store_scatter(lut_v, [val + j * 128], w)

    bufs = (buf0, buf1)
    sems = (sem0, sem1)
    xbufs = (x0, x1)
    xsems = (xsem0, xsem1)

    def compute_panel(x_ref, buf, n_groups, unroll):
        @plsc.parallel_loop(0, n_groups, unroll=unroll)
        def group(q):
            off = pl.multiple_of(q * _L, _L)
            xv = x_ref[pl.ds(off, _L)]
            for j in range(_D):
                buf[j, pl.ds(off, _L)] = plsc.load_gather(lut_v, [xv + j * 128])

    pltpu.async_copy(x_hbm.at[pl.ds(wid * _W, _W)], xbufs[0], xsems[0])

    def pair_body(i, carry):
        for b in range(2):
            g = wid + (2 * i + b) * _NW

            @pl.when(g < _GF)
            def _():
                pltpu.make_async_copy(
                    x_hbm.at[pl.ds(g * _W, _W)], xbufs[b], xsems[b]
                ).wait()

                @pl.when(g + _NW < _GF)
                def _():
                    pltpu.async_copy(
                        x_hbm.at[pl.ds((g + _NW) * _W, _W)],
                        xbufs[1 - b], xsems[1 - b],
                    )

                @pl.when(i >= 1)
                def _():
                    pltpu.make_async_copy(
                        bufs[b],
                        out_hbm.at[:, pl.ds((g - 2 * _NW) * _W, _W)],
                        sems[b],
                    ).wait()

                compute_panel(xbufs[b], bufs[b], _W // _L, 8)
                pltpu.async_copy(bufs[b], out_hbm.at[:, pl.ds(g * _W, _W)],
                                 sems[b])

        return carry

    lax.fori_loop(0, (_GF + 2 * _NW - 1) // (2 * _NW), pair_body, 0)

    @pl.when(wid == _NW - 1)
    def _():
        base = _GF * _W
        pltpu.sync_copy(x_hbm.at[pl.ds(base, _TAIL)], tx_v)
        compute_panel(tx_v, tbuf, _TAIL // _L, 4)
        pltpu.sync_copy(tbuf, out_hbm.at[:, pl.ds(base, _TAIL)])

    for b in range(2):
        pltpu.make_async_copy(bufs[b], out_hbm.at[:, pl.ds(b * _W, _W)],
                              sems[b]).wait()


@jax.jit
def _sc_embed(x, table):
    mesh = plsc.VectorSubcoreMesh(core_axis_name="c", subcore_axis_name="s")
    fn = pl.kernel(
        _body,
        out_type=jax.ShapeDtypeStruct((_D, _N), jnp.float32),
        mesh=mesh,
        scratch_types=[
            pltpu.VMEM((_D * 10,), jnp.float32),
            pltpu.VMEM((_D * _V,), jnp.float32),
            pltpu.VMEM((_W,), jnp.int32),
            pltpu.VMEM((_W,), jnp.int32),
            pltpu.VMEM((_D, _W), jnp.float32),
            pltpu.VMEM((_D, _W), jnp.float32),
            pltpu.VMEM((_D, _TAIL), jnp.float32),
            pltpu.VMEM((_TAIL,), jnp.int32),
            pltpu.SemaphoreType.DMA,
            pltpu.SemaphoreType.DMA,
            pltpu.SemaphoreType.DMA,
            pltpu.SemaphoreType.DMA,
        ],
        compiler_params=pltpu.CompilerParams(
            needs_layout_passes=False, use_tc_tiling_on_sc=True
        ),
    )
    out_t = fn(x, table.reshape(-1))
    return out_t.T


def kernel(x, table):
    return _sc_embed(x, table)


# speedup vs baseline: 41.7177x; 1.1493x over previous
import jax
import jax.numpy as jnp
from jax import lax
from jax.experimental import pallas as pl
from jax.experimental.pallas import tpu as pltpu
from jax.experimental.pallas import tpu_sc as plsc

_BUCKETS = (1, 2, 3, 4, 5, 8, 16, 32, 64)

_N = 1_000_000
_D = 20
_V = 128
_W = 1024
_GF = _N // _W
_TAIL = _N - _GF * _W
_L = 16
_NW = 32
_NB = 3


def _bucketize(val):
    idx = jnp.zeros((_L,), jnp.int32)
    for b in _BUCKETS:
        idx = idx + jnp.where(val >= b, 1, 0)
    return idx


_GATHER_DNUMS = lax.GatherDimensionNumbers(
    offset_dims=(), collapsed_slice_dims=(0,), start_index_map=(0,))


def _vperm(vals, idx):
    return lax.gather(vals, idx[:, None], _GATHER_DNUMS, slice_sizes=(1,),
                      mode=lax.GatherScatterMode.PROMISE_IN_BOUNDS)


def _body(x_hbm, tflat_hbm, out_hbm, tflat_v, vmap_v, x0, x1, x2, buf0,
          buf1, buf2, tbuf, tx_v, sem0, sem1, sem2, xsem0, xsem1, xsem2):
    wid = lax.axis_index("s") * 2 + lax.axis_index("c")

    pltpu.sync_copy(tflat_hbm, tflat_v)

    for g in range(_V // _L):
        val = lax.iota(jnp.int32, _L) + g * _L
        plsc.store_scatter(vmap_v, [val], _bucketize(val))

    rowv = jnp.minimum(lax.iota(jnp.int32, _L), 9) * _D
    tcols = [plsc.load_gather(tflat_v, [rowv + j]) for j in range(_D)]

    bufs = (buf0, buf1, buf2)
    sems = (sem0, sem1, sem2)
    xbufs = (x0, x1, x2)
    xsems = (xsem0, xsem1, xsem2)

    def compute_panel(x_ref, buf, n_groups, unroll):
        def group(q, carry):
            for u in range(unroll):
                off = pl.multiple_of((q * unroll + u) * _L, _L)
                xv = x_ref[pl.ds(off, _L)]
                idxv = plsc.load_gather(vmap_v, [xv])
                for j in range(_D):
                    buf[j, pl.ds(off, _L)] = _vperm(tcols[j], idxv)
            return carry
        lax.fori_loop(0, n_groups // unroll, group, 0)

    pltpu.async_copy(x_hbm.at[pl.ds(wid * _W, _W)], xbufs[0], xsems[0])

    def ring_body(i, carry):
        for b in range(_NB):
            g = wid + (_NB * i + b) * _NW
            xb = b

            @pl.when(g < _GF)
            def _():
                pltpu.make_async_copy(
                    x_hbm.at[pl.ds(g * _W, _W)], xbufs[xb], xsems[xb]
                ).wait()

                @pl.when(g + _NW < _GF)
                def _():
                    pltpu.async_copy(
                        x_hbm.at[pl.ds((g + _NW) * _W, _W)],
                        xbufs[(xb + 1) % _NB], xsems[(xb + 1) % _NB],
                    )

                @pl.when(i >= 1)
                def _():
                    pltpu.make_async_copy(
                        bufs[b],
                        out_hbm.at[:, pl.ds((g - _NB * _NW) * _W, _W)],
                        sems[b],
                    ).wait()

                compute_panel(xbufs[xb], bufs[b], _W // _L, 8)
                pltpu.async_copy(bufs[b], out_hbm.at[:, pl.ds(g * _W, _W)],
                                 sems[b])

        return carry

    lax.fori_loop(0, (_GF + _NB * _NW - 1) // (_NB * _NW), ring_body, 0)

    @pl.when(wid == _NW - 1)
    def _():
        base = _GF * _W
        pltpu.sync_copy(x_hbm.at[pl.ds(base, _TAIL)], tx_v)
        compute_panel(tx_v, tbuf, _TAIL // _L, 4)
        pltpu.sync_copy(tbuf, out_hbm.at[:, pl.ds(base, _TAIL)])

    for b in range(_NB):
        pltpu.make_async_copy(bufs[b], out_hbm.at[:, pl.ds(b * _W, _W)],
                              sems[b]).wait()


@jax.jit
def _sc_embed(x, table):
    mesh = plsc.VectorSubcoreMesh(core_axis_name="c", subcore_axis_name="s")
    fn = pl.kernel(
        _body,
        out_type=jax.ShapeDtypeStruct((_D, _N), jnp.float32),
        mesh=mesh,
        scratch_types=[
            pltpu.VMEM((_D * 10,), jnp.float32),
            pltpu.VMEM((_V,), jnp.int32),
            pltpu.VMEM((_W,), jnp.int32),
            pltpu.VMEM((_W,), jnp.int32),
            pltpu.VMEM((_W,), jnp.int32),
            pltpu.VMEM((_D, _W), jnp.float32),
            pltpu.VMEM((_D, _W), jnp.float32),
            pltpu.VMEM((_D, _W), jnp.float32),
            pltpu.VMEM((_D, _TAIL), jnp.float32),
            pltpu.VMEM((_TAIL,), jnp.int32),
            pltpu.SemaphoreType.DMA,
            pltpu.SemaphoreType.DMA,
            pltpu.SemaphoreType.DMA,
            pltpu.SemaphoreType.DMA,
            pltpu.SemaphoreType.DMA,
            pltpu.SemaphoreType.DMA,
        ],
        compiler_params=pltpu.CompilerParams(
            needs_layout_passes=False, use_tc_tiling_on_sc=True
        ),
    )
    out_t = fn(x, table.reshape(-1))
    return out_t.T


def kernel(x, table):
    return _sc_embed(x, table)
